# baseline (speedup 1.0000x reference)
import functools

import jax
import jax.numpy as jnp
from jax import lax
from jax.experimental import pallas as pl
from jax.experimental.pallas import tpu as pltpu
from jax.experimental.pallas import tpu_sc as plsc

N = 10000
E = 320000
HALF = 5000
HALFP = 5120
NP = 10240
RB = 1024
GRID = NP // RB
F32 = jnp.float32
I32 = jnp.int32


def _mm_bias_body(x_ref, w_ref, b_ref, o_ref):
    o_ref[...] = (
        jnp.dot(x_ref[...], w_ref[...], preferred_element_type=F32) + b_ref[...]
    )


def _mm_bias(x, wt, b2):
    k, m = wt.shape
    return pl.pallas_call(
        _mm_bias_body,
        grid=(GRID,),
        in_specs=[
            pl.BlockSpec((RB, k), lambda i: (i, 0)),
            pl.BlockSpec((k, m), lambda i: (0, 0)),
            pl.BlockSpec((1, m), lambda i: (0, 0)),
        ],
        out_specs=pl.BlockSpec((RB, m), lambda i: (i, 0)),
        out_shape=jax.ShapeDtypeStruct((NP, m), F32),
    )(x, wt, b2)


def _g0_body(h_ref, deg_ref, g_ref, dis_ref):
    deg = deg_ref[0, :] + 1.0
    dis = lax.rsqrt(deg)
    i = pl.program_id(0)
    col = lax.broadcasted_iota(I32, (RB,), 0) + i * RB
    dis = jnp.where(col == NP - 1, 0.0, dis)[:, None]
    dis_ref[...] = dis
    g_ref[...] = dis * h_ref[...]


def _g0(h, deg):
    return pl.pallas_call(
        _g0_body,
        grid=(GRID,),
        in_specs=[
            pl.BlockSpec((RB, 256), lambda i: (i, 0)),
            pl.BlockSpec((1, RB), lambda i: (0, i)),
        ],
        out_specs=[
            pl.BlockSpec((RB, 256), lambda i: (i, 0)),
            pl.BlockSpec((RB, 1), lambda i: (i, 0)),
        ],
        out_shape=[
            jax.ShapeDtypeStruct((NP, 256), F32),
            jax.ShapeDtypeStruct((NP, 1), F32),
        ],
    )(h, deg)


def _layer_body(acc_ref, g_ref, dis_ref, w_ref, b_ref, go_ref):
    dis = dis_ref[...]
    h = jnp.tanh(dis * (acc_ref[...] + g_ref[...]))
    go_ref[...] = dis * (
        jnp.dot(h, w_ref[...], preferred_element_type=F32) + b_ref[...]
    )


def _layer(acc, g, dis, wt, b2):
    k, m = wt.shape
    return pl.pallas_call(
        _layer_body,
        grid=(GRID,),
        in_specs=[
            pl.BlockSpec((RB, k), lambda i: (i, 0)),
            pl.BlockSpec((RB, k), lambda i: (i, 0)),
            pl.BlockSpec((RB, 1), lambda i: (i, 0)),
            pl.BlockSpec((k, m), lambda i: (0, 0)),
            pl.BlockSpec((1, m), lambda i: (0, 0)),
        ],
        out_specs=pl.BlockSpec((RB, m), lambda i: (i, 0)),
        out_shape=jax.ShapeDtypeStruct((NP, m), F32),
    )(acc, g, dis, wt, b2)


def _final_body(acc_ref, g_ref, dis_ref, w_ref, b_ref, o_ref):
    dis = dis_ref[...]
    h = jnp.tanh(dis * (acc_ref[...] + g_ref[...]))
    o_ref[...] = jnp.dot(h, w_ref[...], preferred_element_type=F32) + b_ref[...]


def _final(acc, g, dis, wt, b2):
    k, m = wt.shape
    return pl.pallas_call(
        _final_body,
        grid=(GRID,),
        in_specs=[
            pl.BlockSpec((RB, k), lambda i: (i, 0)),
            pl.BlockSpec((RB, k), lambda i: (i, 0)),
            pl.BlockSpec((RB, 1), lambda i: (i, 0)),
            pl.BlockSpec((k, m), lambda i: (0, 0)),
            pl.BlockSpec((1, m), lambda i: (0, 0)),
        ],
        out_specs=pl.BlockSpec((RB, m), lambda i: (i, 0)),
        out_shape=jax.ShapeDtypeStruct((NP, m), F32),
    )(acc, g, dis, wt, b2)


_VSM = plsc.VectorSubcoreMesh(core_axis_name="c", subcore_axis_name="s")
NTILE = 16
NODES_T = NP // 32
R_MAX = 64
E_LEFT = 4096
ZROW = NP - 1


@functools.partial(
    pl.kernel,
    out_type=jax.ShapeDtypeStruct((NP, 256), F32),
    mesh=_VSM,
    scratch_types=[
        pltpu.VMEM((NODES_T,), I32),
        pltpu.VMEM((NODES_T, 256), F32),
        pltpu.SemaphoreType.DMA,
    ],
    compiler_params=pltpu.CompilerParams(use_tc_tiling_on_sc=False),
)
def _gadd_kernel(g_hbm, idx_hbm, out_hbm, idxb, acc, sem):
    c = lax.axis_index("c")
    s = lax.axis_index("s")
    t = c * NTILE + s
    base = pl.multiple_of(t * NODES_T, 16)
    pltpu.sync_copy(idx_hbm.at[pl.ds(base, NODES_T)], idxb)
    pltpu.async_copy(g_hbm.at[idxb], acc, sem).wait()

    def rbody(r, carry):
        ro = pl.multiple_of(r * NP + t * NODES_T, 16)
        pltpu.sync_copy(idx_hbm.at[pl.ds(ro, NODES_T)], idxb)
        pltpu.async_copy(g_hbm.at[idxb], acc, sem, add=True).wait()
        return carry

    lax.fori_loop(1, R_MAX, rbody, 0)
    pltpu.sync_copy(acc, out_hbm.at[pl.ds(base, NODES_T)])


def kernel(x, edge_index, batch, W0, b0, W1, b1, W2, b2, W3, b3, Wout, bout):
    src = edge_index[0]
    dst = edge_index[1]
    srcp = src + jnp.where(src >= HALF, 120, 0).astype(I32)
    dstp = dst + jnp.where(dst >= HALF, 120, 0).astype(I32)

    order = jnp.argsort(dstp)
    dst_s = dstp[order]
    srcp_s = srcp[order]
    bounds = jnp.searchsorted(dst_s, jnp.arange(NP + 1, dtype=I32)).astype(I32)
    starts = bounds[:-1]
    ends = bounds[1:]
    rr = jnp.arange(R_MAX, dtype=I32)[:, None]
    pos = starts[None, :] + rr
    valid = pos < ends[None, :]
    idx_mat = jnp.where(valid, srcp_s[jnp.clip(pos, 0, E - 1)], ZROW)
    idx_flat = idx_mat.reshape(-1)

    ranks = jnp.arange(E, dtype=I32) - starts[dst_s]
    li = jnp.where(ranks >= R_MAX, size=E_LEFT, fill_value=E)[0]
    in_e = li < E
    lic = jnp.clip(li, 0, E - 1)
    src_lf = jnp.where(in_e, srcp_s[lic], ZROW)
    dst_lf = jnp.where(in_e, dst_s[lic], ZROW)

    src_sorted = jnp.sort(srcp)
    sb = jnp.searchsorted(src_sorted, jnp.arange(NP + 1, dtype=I32))
    deg = (sb[1:] - sb[:-1]).astype(F32)[None, :]

    z = jnp.zeros((HALFP - HALF, 128), F32)
    xp = jnp.concatenate([x[:HALF], z, x[HALF:], z], axis=0)

    h0 = _mm_bias(xp, W0.T, b0[None, :])
    g, dis = _g0(h0, deg)

    for wt, b in ((W1, b1), (W2, b2), (W3, b3)):
        acc = _gadd_kernel(g, idx_flat)
        acc = acc.at[dst_lf].add(g[src_lf])
        g = _layer(acc, g, dis, wt.T, b[None, :])

    acc = _gadd_kernel(g, idx_flat)
    acc = acc.at[dst_lf].add(g[src_lf])
    out = _final(acc, g, dis, Wout.T, bout[None, :])
    return jnp.concatenate([out[:HALF], out[HALFP : HALFP + HALF]], axis=0)


# speedup vs baseline: 2.0270x; 2.0270x over previous
import functools

import jax
import jax.numpy as jnp
from jax import lax
from jax.experimental import pallas as pl
from jax.experimental.pallas import tpu as pltpu
from jax.experimental.pallas import tpu_sc as plsc

N = 10000
E = 320000
HALF = 5000
HALFP = 5120
NP = 10240
RB = 1024
GRID = NP // RB
F32 = jnp.float32
I32 = jnp.int32


def _mm_bias_body(x_ref, w_ref, b_ref, o_ref):
    o_ref[...] = (
        jnp.dot(x_ref[...], w_ref[...], preferred_element_type=F32) + b_ref[...]
    )


def _mm_bias(x, wt, b2):
    k, m = wt.shape
    return pl.pallas_call(
        _mm_bias_body,
        grid=(GRID,),
        in_specs=[
            pl.BlockSpec((RB, k), lambda i: (i, 0)),
            pl.BlockSpec((k, m), lambda i: (0, 0)),
            pl.BlockSpec((1, m), lambda i: (0, 0)),
        ],
        out_specs=pl.BlockSpec((RB, m), lambda i: (i, 0)),
        out_shape=jax.ShapeDtypeStruct((NP, m), F32),
    )(x, wt, b2)


def _g0_body(h_ref, deg_ref, g_ref, dis_ref):
    deg = deg_ref[0, :] + 1.0
    dis = lax.rsqrt(deg)
    i = pl.program_id(0)
    col = lax.broadcasted_iota(I32, (RB,), 0) + i * RB
    dis = jnp.where(col == NP - 1, 0.0, dis)[:, None]
    dis_ref[...] = dis
    g_ref[...] = dis * h_ref[...]


def _g0(h, deg):
    return pl.pallas_call(
        _g0_body,
        grid=(GRID,),
        in_specs=[
            pl.BlockSpec((RB, 256), lambda i: (i, 0)),
            pl.BlockSpec((1, RB), lambda i: (0, i)),
        ],
        out_specs=[
            pl.BlockSpec((RB, 256), lambda i: (i, 0)),
            pl.BlockSpec((RB, 1), lambda i: (i, 0)),
        ],
        out_shape=[
            jax.ShapeDtypeStruct((NP, 256), F32),
            jax.ShapeDtypeStruct((NP, 1), F32),
        ],
    )(h, deg)


def _layer_body(acc_ref, g_ref, dis_ref, w_ref, b_ref, go_ref):
    dis = dis_ref[...]
    h = jnp.tanh(dis * (acc_ref[...] + g_ref[...]))
    go_ref[...] = dis * (
        jnp.dot(h, w_ref[...], preferred_element_type=F32) + b_ref[...]
    )


def _layer(acc, g, dis, wt, b2):
    k, m = wt.shape
    return pl.pallas_call(
        _layer_body,
        grid=(GRID,),
        in_specs=[
            pl.BlockSpec((RB, k), lambda i: (i, 0)),
            pl.BlockSpec((RB, k), lambda i: (i, 0)),
            pl.BlockSpec((RB, 1), lambda i: (i, 0)),
            pl.BlockSpec((k, m), lambda i: (0, 0)),
            pl.BlockSpec((1, m), lambda i: (0, 0)),
        ],
        out_specs=pl.BlockSpec((RB, m), lambda i: (i, 0)),
        out_shape=jax.ShapeDtypeStruct((NP, m), F32),
    )(acc, g, dis, wt, b2)


def _final_body(acc_ref, g_ref, dis_ref, w_ref, b_ref, o_ref):
    dis = dis_ref[...]
    h = jnp.tanh(dis * (acc_ref[...] + g_ref[...]))
    o_ref[...] = jnp.dot(h, w_ref[...], preferred_element_type=F32) + b_ref[...]


def _final(acc, g, dis, wt, b2):
    k, m = wt.shape
    return pl.pallas_call(
        _final_body,
        grid=(GRID,),
        in_specs=[
            pl.BlockSpec((RB, k), lambda i: (i, 0)),
            pl.BlockSpec((RB, k), lambda i: (i, 0)),
            pl.BlockSpec((RB, 1), lambda i: (i, 0)),
            pl.BlockSpec((k, m), lambda i: (0, 0)),
            pl.BlockSpec((1, m), lambda i: (0, 0)),
        ],
        out_specs=pl.BlockSpec((RB, m), lambda i: (i, 0)),
        out_shape=jax.ShapeDtypeStruct((NP, m), F32),
    )(acc, g, dis, wt, b2)


_VSM = plsc.VectorSubcoreMesh(core_axis_name="c", subcore_axis_name="s")
NTILE = 16
NODES_T = NP // 32
R_MAX = 48
E_LEFT = 8192
CH = 80
ZROW = NP - 1


@functools.partial(
    pl.kernel,
    out_type=jax.ShapeDtypeStruct((NP, 256), F32),
    mesh=_VSM,
    scratch_types=[
        pltpu.VMEM((NODES_T,), I32),
        pltpu.VMEM((CH,), I32),
        pltpu.VMEM((CH, 256), F32),
        pltpu.VMEM((NODES_T, 256), F32),
        pltpu.SemaphoreType.DMA,
    ],
)
def _gadd_kernel(g_hbm, idx_hbm, out_hbm, idxb, idxc, rows, acc, sem):
    c = lax.axis_index("c")
    s = lax.axis_index("s")
    t = c * NTILE + s
    base = pl.multiple_of(t * NODES_T, 16)
    pltpu.sync_copy(idx_hbm.at[pl.ds(base, NODES_T)], idxb)
    pltpu.async_copy(g_hbm.at[idxb], acc, sem).wait()

    def rbody(r, carry):
        ro = pl.multiple_of(r * NP + t * NODES_T, 16)
        pltpu.sync_copy(idx_hbm.at[pl.ds(ro, NODES_T)], idxb)
        for cc in range(NODES_T // CH):
            co = cc * CH
            for q in range(CH // 16):
                qo = pl.multiple_of(co + q * 16, 16)
                idxc[pl.ds(q * 16, 16)] = idxb[pl.ds(qo, 16)]
            pltpu.async_copy(g_hbm.at[idxc], rows, sem).wait()

            def arow(i, carry2):
                for k in range(16):
                    ko = pl.multiple_of(k * 16, 16)
                    v = rows[i, pl.ds(ko, 16)]
                    a = acc[co + i, pl.ds(ko, 16)]
                    acc[co + i, pl.ds(ko, 16)] = a + v
                return carry2

            lax.fori_loop(0, CH, arow, 0)
        return carry

    lax.fori_loop(1, R_MAX, rbody, 0)
    pltpu.sync_copy(acc, out_hbm.at[pl.ds(base, NODES_T)])


def kernel(x, edge_index, batch, W0, b0, W1, b1, W2, b2, W3, b3, Wout, bout):
    src = edge_index[0]
    dst = edge_index[1]
    srcp = src + jnp.where(src >= HALF, 120, 0).astype(I32)
    dstp = dst + jnp.where(dst >= HALF, 120, 0).astype(I32)

    order = jnp.argsort(dstp)
    dst_s = dstp[order]
    srcp_s = srcp[order]
    bounds = jnp.searchsorted(dst_s, jnp.arange(NP + 1, dtype=I32)).astype(I32)
    starts = bounds[:-1]
    ends = bounds[1:]
    rr = jnp.arange(R_MAX, dtype=I32)[:, None]
    pos = starts[None, :] + rr
    valid = pos < ends[None, :]
    idx_mat = jnp.where(valid, srcp_s[jnp.clip(pos, 0, E - 1)], ZROW)
    idx_flat = idx_mat.reshape(-1)

    ranks = jnp.arange(E, dtype=I32) - starts[dst_s]
    li = jnp.where(ranks >= R_MAX, size=E_LEFT, fill_value=E)[0]
    in_e = li < E
    lic = jnp.clip(li, 0, E - 1)
    src_lf = jnp.where(in_e, srcp_s[lic], ZROW)
    dst_lf = jnp.where(in_e, dst_s[lic], ZROW)

    src_sorted = jnp.sort(srcp)
    sb = jnp.searchsorted(src_sorted, jnp.arange(NP + 1, dtype=I32))
    deg = (sb[1:] - sb[:-1]).astype(F32)[None, :]

    z = jnp.zeros((HALFP - HALF, 128), F32)
    xp = jnp.concatenate([x[:HALF], z, x[HALF:], z], axis=0)

    h0 = _mm_bias(xp, W0.T, b0[None, :])
    g, dis = _g0(h0, deg)

    for wt, b in ((W1, b1), (W2, b2), (W3, b3)):
        acc = _gadd_kernel(g, idx_flat)
        acc = acc.at[dst_lf].add(g[src_lf])
        g = _layer(acc, g, dis, wt.T, b[None, :])

    acc = _gadd_kernel(g, idx_flat)
    acc = acc.at[dst_lf].add(g[src_lf])
    out = _final(acc, g, dis, Wout.T, bout[None, :])
    return jnp.concatenate([out[:HALF], out[HALFP : HALFP + HALF]], axis=0)
